# Initial kernel scaffold; baseline (speedup 1.0000x reference)
import jax
import jax.numpy as jnp
from jax.experimental import pallas as pl


def kernel(features):
    raise NotImplementedError("write your pallas kernel here")


# speedup vs baseline: 5.8059x; 5.8059x over previous
import functools

import jax
import jax.numpy as jnp
from jax.experimental import pallas as pl

_CLUSTER_NUM = 512
_MAX_ITER = 10
_TOL = 0.005


def _kmeans_body(feat_ref, perm_ref, out_ref):
    x = feat_ref[0]
    perm = perm_ref[0, 0]
    n = x.shape[0]
    k = _CLUSTER_NUM

    iota_kn = jax.lax.broadcasted_iota(jnp.int32, (k, n), 1)
    sel = (iota_kn == perm[:, None]).astype(jnp.float32)
    centers0 = jax.lax.dot_general(
        sel, x, (((1,), (0,)), ((), ())),
        precision=jax.lax.Precision.HIGHEST)

    xx = jnp.sum(x * x, axis=1, keepdims=True)
    iota_nk = jax.lax.broadcasted_iota(jnp.int32, (n, k), 1)

    def cond(state):
        i, _, _, done = state
        return (i < _MAX_ITER) & jnp.logical_not(done)

    def body(state):
        i, centers, _, done = state
        cc = jnp.sum(centers * centers, axis=1)
        xc = jax.lax.dot_general(
            x, centers, (((1,), (1,)), ((), ())))
        d2 = xx - 2.0 * xc + cc[None, :]
        dist = jnp.sqrt(jnp.maximum(d2, 0.0))
        mind = jnp.min(dist, axis=1, keepdims=True)
        assign = jnp.min(
            jnp.where(dist == mind, iota_nk, jnp.int32(k)),
            axis=1, keepdims=True)
        onehot = (iota_nk == assign).astype(jnp.float32)
        sums = jax.lax.dot_general(
            onehot, x, (((0,), (0,)), ((), ())),
            precision=jax.lax.Precision.HIGHEST)
        counts = jnp.sum(onehot, axis=0)
        new_centers = jnp.where(
            counts[:, None] > 0,
            sums / jnp.maximum(counts[:, None], 1.0), centers)
        shift = jnp.sum(jnp.sqrt(jnp.sum((new_centers - centers) ** 2,
                                         axis=1)))
        return (i + 1, new_centers, assign, shift < _TOL * n)

    init = (jnp.int32(0), centers0,
            jnp.zeros((n, 1), jnp.int32), jnp.array(False))
    _, _, assign, _ = jax.lax.while_loop(cond, body, init)
    out_ref[0] = assign


@functools.partial(jax.jit, static_argnames=())
def kernel(features):
    b, n, d = features.shape
    k = _CLUSTER_NUM
    perms = [
        jax.random.permutation(
            jax.random.fold_in(jax.random.key(42), i), n)[:k]
        for i in range(b)
    ]
    perm = jnp.stack(perms).astype(jnp.int32).reshape(b, 1, k)

    out = pl.pallas_call(
        _kmeans_body,
        grid=(b,),
        in_specs=[
            pl.BlockSpec((1, n, d), lambda i: (i, 0, 0)),
            pl.BlockSpec((1, 1, k), lambda i: (i, 0, 0)),
        ],
        out_specs=pl.BlockSpec((1, n, 1), lambda i: (i, 0, 0)),
        out_shape=jax.ShapeDtypeStruct((b, n, 1), jnp.int32),
    )(features, perm)
    return out.reshape(b, n)


# speedup vs baseline: 7.0487x; 1.2141x over previous
import functools

import jax
import jax.numpy as jnp
from jax.experimental import pallas as pl

_CLUSTER_NUM = 512
_MAX_ITER = 10
_TOL = 0.005


def _kmeans_body(feat_ref, cinit_ref, out_ref):
    n = feat_ref.shape[1]
    k = _CLUSTER_NUM
    x2 = [feat_ref[0], feat_ref[1]]
    iota_nk = jax.lax.broadcasted_iota(jnp.int32, (n, k), 1)
    ones_n = jnp.ones((n, 1), jnp.float32)
    xaug2 = [jnp.concatenate([x, ones_n], axis=1) for x in x2]
    xx2 = [jnp.sum(x * x, axis=1, keepdims=True) for x in x2]

    def step(x, xaug, xx, centers):
        cc = jnp.sum(centers * centers, axis=1)
        xc = jax.lax.dot_general(
            x, centers, (((1,), (1,)), ((), ())))
        d2 = xx - 2.0 * xc + cc[None, :]
        dist = jnp.sqrt(jnp.maximum(d2, 0.0))
        mind = jnp.min(dist, axis=1, keepdims=True)
        assign = jnp.min(
            jnp.where(dist == mind, iota_nk, jnp.int32(k)),
            axis=1, keepdims=True)
        onehot = (iota_nk == assign).astype(jnp.float32)
        sums_aug = jax.lax.dot_general(
            onehot, xaug, (((0,), (0,)), ((), ())),
            precision=jax.lax.Precision.HIGHEST)
        sums = sums_aug[:, :-1]
        counts = sums_aug[:, -1:]
        new_centers = jnp.where(
            counts > 0, sums / jnp.maximum(counts, 1.0), centers)
        shift = jnp.sum(jnp.sqrt(jnp.sum((new_centers - centers) ** 2,
                                         axis=1)))
        return new_centers, assign, shift < _TOL * n

    def cond(state):
        i, _, _, done = state
        return (i < _MAX_ITER) & jnp.logical_not(done[0] & done[1])

    def body(state):
        i, centers2, assign2, done2 = state
        new_centers2, new_assign2, new_done2 = [], [], []
        for j in range(2):
            nc, na, nd = step(x2[j], xaug2[j], xx2[j], centers2[j])
            new_centers2.append(jnp.where(done2[j], centers2[j], nc))
            new_assign2.append(jnp.where(done2[j], assign2[j], na))
            new_done2.append(done2[j] | nd)
        return (i + 1, tuple(new_centers2), tuple(new_assign2),
                tuple(new_done2))

    init = (jnp.int32(0), (cinit_ref[0], cinit_ref[1]),
            (jnp.zeros((n, 1), jnp.int32),) * 2,
            (jnp.array(False),) * 2)
    _, _, assign2, _ = jax.lax.while_loop(cond, body, init)
    out_ref[0, :, 0:1] = assign2[0]
    out_ref[0, :, 1:2] = assign2[1]


@functools.partial(jax.jit, static_argnames=())
def kernel(features):
    b, n, d = features.shape
    k = _CLUSTER_NUM
    centers0 = jnp.stack([
        features[i][jax.random.permutation(
            jax.random.fold_in(jax.random.key(42), i), n)[:k]]
        for i in range(b)
    ])

    out = pl.pallas_call(
        _kmeans_body,
        grid=(b // 2,),
        in_specs=[
            pl.BlockSpec((2, n, d), lambda i: (i, 0, 0)),
            pl.BlockSpec((2, k, d), lambda i: (i, 0, 0)),
        ],
        out_specs=pl.BlockSpec((1, n, 2), lambda i: (i, 0, 0)),
        out_shape=jax.ShapeDtypeStruct((b // 2, n, 2), jnp.int32),
    )(features, centers0)
    return out.transpose(0, 2, 1).reshape(b, n)
